# baseline (speedup 1.0000x reference)
import jax
import jax.numpy as jnp
from jax import lax
from jax.experimental import pallas as pl
from jax.experimental.pallas import tpu as pltpu, tpu_sc as plsc

N_NODES = 100000
N_FEATS = 9
VOCAB = 100
HIDDEN = 256
NPAIR = 5
PAIRB = VOCAB * VOCAB

BLK = 256
NB_TC = 156
M_TC = NB_TC * BLK
M_SC = N_NODES - M_TC

NC = 2
NS = 16
NW = NC * NS

C = 32
FLAT = C * N_FEATS
NCHUNKS = M_SC // C
T = (NCHUNKS + NW - 1) // NW
NITER = (T + 1) // 2

_mesh = plsc.VectorSubcoreMesh(core_axis_name="c", subcore_axis_name="s")


def _stage_x(xc_hbm, xbuf, g, sem):
    return pltpu.async_copy(xc_hbm.at[pl.ds(g * FLAT, FLAT)], xbuf, sem)


def _compute_idx(xbuf, idxs):
    i01, i234 = idxs
    for p in range(4):
        ref, off = (i01, p * C) if p < 2 else (i234, (p - 2) * C)
        for k in range(2):
            da = pl.ds((2 * p) * C + 16 * k, 16)
            db = pl.ds((2 * p + 1) * C + 16 * k, 16)
            ref[pl.ds(off + 16 * k, 16)] = (
                xbuf[da] * VOCAB + xbuf[db] + p * PAIRB)
    for k in range(2):
        d8 = pl.ds(8 * C + 16 * k, 16)
        i234[pl.ds(2 * C + 16 * k, 16)] = xbuf[d8] + 4 * PAIRB


def _issue_gathers(tbl_hbm, idxs, rows, sem):
    i01, i234 = idxs
    return [
        pltpu.async_copy(tbl_hbm.at[i01], rows.at[pl.ds(0, 2 * C)], sem),
        pltpu.async_copy(tbl_hbm.at[i234], rows.at[pl.ds(2 * C, 3 * C)],
                         sem),
    ]


def _wait_gathers(tbl_hbm, idxs, rows, sem):
    i01, i234 = idxs
    pltpu.make_async_copy(tbl_hbm.at[i01],
                          rows.at[pl.ds(0, 2 * C)], sem).wait()
    pltpu.make_async_copy(tbl_hbm.at[i234],
                          rows.at[pl.ds(2 * C, 3 * C)], sem).wait()


def _sum_chunk(rows, out_v):
    def node_sum(m, carry):
        for n in (2 * m, 2 * m + 1):
            for k in range(HIDDEN // 16):
                d = pl.ds(16 * k, 16)
                t0 = rows[n, d] + rows[C + n, d]
                t1 = rows[2 * C + n, d] + rows[3 * C + n, d]
                out_v[n, d] = t0 + t1 + rows[4 * C + n, d]
        return carry

    lax.fori_loop(0, C // 2, node_sum, 0)


def _body(xc_hbm, tbl_hbm, out_hbm, xA, xB, iA01, iA234, iB01, iB234,
          rowsA, rowsB, outA, outB, sem_x, sem_g, sem_oA, sem_oB):
    c = lax.axis_index("c")
    s = lax.axis_index("s")
    wid = s * NC + c
    g_start = wid * T
    idxsA = (iA01, iA234)
    idxsB = (iB01, iB234)

    _stage_x(xc_hbm, xA, g_start, sem_x).wait()
    _compute_idx(xA, idxsA)
    _issue_gathers(tbl_hbm, idxsA, rowsA, sem_g)

    def step(i, carry):
        e = 2 * i
        ge = g_start + e
        go = ge + 1
        gn = ge + 2
        ve = ge < NCHUNKS
        vo = go < NCHUNKS
        vn = gn < NCHUNKS

        @pl.when(vo)
        def _():
            _stage_x(xc_hbm, xB, go, sem_x)

        @pl.when(ve)
        def _():
            _wait_gathers(tbl_hbm, idxsA, rowsA, sem_g)

        @pl.when(vo)
        def _():
            pltpu.make_async_copy(xc_hbm.at[pl.ds(go * FLAT, FLAT)],
                                  xB, sem_x).wait()
            _compute_idx(xB, idxsB)
            _issue_gathers(tbl_hbm, idxsB, rowsB, sem_g)

        @pl.when(ve)
        def _():
            @pl.when(i > 0)
            def _():
                pltpu.make_async_copy(
                    outA, out_hbm.at[pl.ds(0, C)], sem_oA).wait()
            _sum_chunk(rowsA, outA)
            pltpu.async_copy(outA, out_hbm.at[pl.ds(ge * C, C)], sem_oA)

        @pl.when(vn)
        def _():
            _stage_x(xc_hbm, xA, gn, sem_x)

        @pl.when(vo)
        def _():
            _wait_gathers(tbl_hbm, idxsB, rowsB, sem_g)

        @pl.when(vn)
        def _():
            pltpu.make_async_copy(xc_hbm.at[pl.ds(gn * FLAT, FLAT)],
                                  xA, sem_x).wait()
            _compute_idx(xA, idxsA)
            _issue_gathers(tbl_hbm, idxsA, rowsA, sem_g)

        @pl.when(vo)
        def _():
            @pl.when(i > 0)
            def _():
                pltpu.make_async_copy(
                    outB, out_hbm.at[pl.ds(0, C)], sem_oB).wait()
            _sum_chunk(rowsB, outB)
            pltpu.async_copy(outB, out_hbm.at[pl.ds(go * C, C)], sem_oB)
        return carry

    lax.fori_loop(0, NITER, step, 0)

    pltpu.make_async_copy(outA, out_hbm.at[pl.ds(0, C)], sem_oA).wait()
    pltpu.make_async_copy(outB, out_hbm.at[pl.ds(0, C)], sem_oB).wait()


_sc_call = pl.kernel(
    _body,
    out_type=jax.ShapeDtypeStruct((M_SC, HIDDEN), jnp.float32),
    mesh=_mesh,
    scratch_types=(
        [pltpu.VMEM((FLAT,), jnp.int32)] * 2
        + [pltpu.VMEM((2 * C,), jnp.int32),
           pltpu.VMEM((3 * C,), jnp.int32),
           pltpu.VMEM((2 * C,), jnp.int32),
           pltpu.VMEM((3 * C,), jnp.int32)]
        + [pltpu.VMEM((NPAIR * C, HIDDEN), jnp.float32)] * 2
        + [pltpu.VMEM((C, HIDDEN), jnp.float32)] * 2
        + [pltpu.SemaphoreType.DMA] * 4
    ),
)


def _tc_body(xg_ref, tbl_ref, out_ref):
    f = pl.program_id(1)
    idx = xg_ref[0, 0, :] + f * VOCAB
    iot = lax.broadcasted_iota(jnp.int32, (BLK, 1024), 1)
    oh = (idx[:, None] == iot).astype(jnp.bfloat16)
    prod = jnp.dot(oh, tbl_ref[...], preferred_element_type=jnp.float32)

    @pl.when(f == 0)
    def _():
        out_ref[...] = prod

    @pl.when(f != 0)
    def _():
        out_ref[...] = out_ref[...] + prod


_tc_call = pl.pallas_call(
    _tc_body,
    grid=(NB_TC, N_FEATS),
    in_specs=[
        pl.BlockSpec((1, 1, BLK), lambda b, f: (f * NB_TC + b, 0, 0)),
        pl.BlockSpec((1024, HIDDEN), lambda b, f: (0, 0)),
    ],
    out_specs=pl.BlockSpec((BLK, HIDDEN), lambda b, f: (b, 0)),
    out_shape=jax.ShapeDtypeStruct((M_TC, HIDDEN), jnp.float32),
    compiler_params=pltpu.CompilerParams(
        dimension_semantics=("parallel", "arbitrary")),
)


def kernel(x, emb):
    pairs = [
        (emb[2 * p][:, None, :] + emb[2 * p + 1][None, :, :]).reshape(
            PAIRB, HIDDEN)
        for p in range(4)
    ]
    tbl = jnp.concatenate(pairs + [emb[8]], axis=0)
    x_sc = x[:M_SC]
    xc = x_sc.T.reshape(N_FEATS, NCHUNKS, C).transpose(1, 0, 2).reshape(-1)
    sc_out = _sc_call(xc, tbl)

    x_tc = x[M_SC:]
    xg = x_tc.T.reshape(N_FEATS * NB_TC, 1, BLK)
    emb_flat = emb.reshape(N_FEATS * VOCAB, HIDDEN)
    tblp = jnp.pad(emb_flat.astype(jnp.bfloat16),
                   ((0, 1024 - N_FEATS * VOCAB), (0, 0)))
    tc_out = _tc_call(xg, tblp)
    return jnp.concatenate([sc_out, tc_out], axis=0)


# speedup vs baseline: 2.3708x; 2.3708x over previous
import jax
import jax.numpy as jnp
from jax import lax
from jax.experimental import pallas as pl
from jax.experimental.pallas import tpu as pltpu, tpu_sc as plsc

N_NODES = 100000
N_FEATS = 9
VOCAB = 100
HIDDEN = 256
NPAIR = 5
PAIRB = VOCAB * VOCAB

BLK = 256
NB_TC = 156
M_TC = NB_TC * BLK
M_SC = N_NODES - M_TC

NC = 2
NS = 16
NW = NC * NS

C = 32
FLAT = C * N_FEATS
NCHUNKS = M_SC // C
T = (NCHUNKS + NW - 1) // NW
NITER = (T + 1) // 2

_mesh = plsc.VectorSubcoreMesh(core_axis_name="c", subcore_axis_name="s")


def _stage_x(xc_hbm, xbuf, g, sem):
    return pltpu.async_copy(xc_hbm.at[pl.ds(g * FLAT, FLAT)], xbuf, sem)


def _compute_idx(xbuf, idxs):
    i01, i234 = idxs
    for p in range(4):
        ref, off = (i01, p * C) if p < 2 else (i234, (p - 2) * C)
        for k in range(2):
            da = pl.ds((2 * p) * C + 16 * k, 16)
            db = pl.ds((2 * p + 1) * C + 16 * k, 16)
            ref[pl.ds(off + 16 * k, 16)] = (
                xbuf[da] * VOCAB + xbuf[db] + p * PAIRB)
    for k in range(2):
        d8 = pl.ds(8 * C + 16 * k, 16)
        i234[pl.ds(2 * C + 16 * k, 16)] = xbuf[d8] + 4 * PAIRB


def _issue_gathers(tbl_hbm, idxs, rows, sem):
    i01, i234 = idxs
    return [
        pltpu.async_copy(tbl_hbm.at[i01], rows.at[pl.ds(0, 2 * C)], sem),
        pltpu.async_copy(tbl_hbm.at[i234], rows.at[pl.ds(2 * C, 3 * C)],
                         sem),
    ]


def _wait_gathers(tbl_hbm, idxs, rows, sem):
    i01, i234 = idxs
    pltpu.make_async_copy(tbl_hbm.at[i01],
                          rows.at[pl.ds(0, 2 * C)], sem).wait()
    pltpu.make_async_copy(tbl_hbm.at[i234],
                          rows.at[pl.ds(2 * C, 3 * C)], sem).wait()


def _sum_chunk(rows, out_v):
    def node_sum(m, carry):
        for n in (2 * m, 2 * m + 1):
            for k in range(HIDDEN // 16):
                d = pl.ds(16 * k, 16)
                t0 = rows[n, d] + rows[C + n, d]
                t1 = rows[2 * C + n, d] + rows[3 * C + n, d]
                out_v[n, d] = t0 + t1 + rows[4 * C + n, d]
        return carry

    lax.fori_loop(0, C // 2, node_sum, 0)


def _body(xc_hbm, tbl_hbm, out_hbm, xA, xB, iA01, iA234, iB01, iB234,
          rowsA, rowsB, outA, outB, sem_x, sem_g, sem_oA, sem_oB):
    c = lax.axis_index("c")
    s = lax.axis_index("s")
    wid = s * NC + c
    g_start = wid * T
    idxsA = (iA01, iA234)
    idxsB = (iB01, iB234)

    _stage_x(xc_hbm, xA, g_start, sem_x).wait()
    _compute_idx(xA, idxsA)
    _issue_gathers(tbl_hbm, idxsA, rowsA, sem_g)

    def step(i, carry):
        e = 2 * i
        ge = g_start + e
        go = ge + 1
        gn = ge + 2
        ve = ge < NCHUNKS
        vo = go < NCHUNKS
        vn = gn < NCHUNKS

        @pl.when(vo)
        def _():
            _stage_x(xc_hbm, xB, go, sem_x)

        @pl.when(ve)
        def _():
            _wait_gathers(tbl_hbm, idxsA, rowsA, sem_g)

        @pl.when(vo)
        def _():
            pltpu.make_async_copy(xc_hbm.at[pl.ds(go * FLAT, FLAT)],
                                  xB, sem_x).wait()
            _compute_idx(xB, idxsB)
            _issue_gathers(tbl_hbm, idxsB, rowsB, sem_g)

        @pl.when(ve)
        def _():
            @pl.when(i > 0)
            def _():
                pltpu.make_async_copy(
                    outA, out_hbm.at[pl.ds(0, C)], sem_oA).wait()
            _sum_chunk(rowsA, outA)
            pltpu.async_copy(outA, out_hbm.at[pl.ds(ge * C, C)], sem_oA)

        @pl.when(vn)
        def _():
            _stage_x(xc_hbm, xA, gn, sem_x)

        @pl.when(vo)
        def _():
            _wait_gathers(tbl_hbm, idxsB, rowsB, sem_g)

        @pl.when(vn)
        def _():
            pltpu.make_async_copy(xc_hbm.at[pl.ds(gn * FLAT, FLAT)],
                                  xA, sem_x).wait()
            _compute_idx(xA, idxsA)
            _issue_gathers(tbl_hbm, idxsA, rowsA, sem_g)

        @pl.when(vo)
        def _():
            @pl.when(i > 0)
            def _():
                pltpu.make_async_copy(
                    outB, out_hbm.at[pl.ds(0, C)], sem_oB).wait()
            _sum_chunk(rowsB, outB)
            pltpu.async_copy(outB, out_hbm.at[pl.ds(go * C, C)], sem_oB)
        return carry

    lax.fori_loop(0, NITER, step, 0)

    pltpu.make_async_copy(outA, out_hbm.at[pl.ds(0, C)], sem_oA).wait()
    pltpu.make_async_copy(outB, out_hbm.at[pl.ds(0, C)], sem_oB).wait()


_sc_call = pl.kernel(
    _body,
    out_type=jax.ShapeDtypeStruct((M_SC, HIDDEN), jnp.float32),
    mesh=_mesh,
    scratch_types=(
        [pltpu.VMEM((FLAT,), jnp.int32)] * 2
        + [pltpu.VMEM((2 * C,), jnp.int32),
           pltpu.VMEM((3 * C,), jnp.int32),
           pltpu.VMEM((2 * C,), jnp.int32),
           pltpu.VMEM((3 * C,), jnp.int32)]
        + [pltpu.VMEM((NPAIR * C, HIDDEN), jnp.float32)] * 2
        + [pltpu.VMEM((C, HIDDEN), jnp.float32)] * 2
        + [pltpu.SemaphoreType.DMA] * 4
    ),
)


def _tc_body(xg_ref, tbl_ref, out_ref):
    iot = lax.broadcasted_iota(jnp.int32, (BLK, 1024), 1)
    oh = None
    for f in range(N_FEATS):
        idx = xg_ref[f, 0, :] + f * VOCAB
        m = idx[:, None] == iot
        oh = m if oh is None else (oh | m)
    out_ref[...] = jnp.dot(oh.astype(jnp.bfloat16), tbl_ref[...],
                           preferred_element_type=jnp.float32)


_tc_call = pl.pallas_call(
    _tc_body,
    grid=(NB_TC,),
    in_specs=[
        pl.BlockSpec((N_FEATS, 1, BLK), lambda b: (b, 0, 0)),
        pl.BlockSpec((1024, HIDDEN), lambda b: (0, 0)),
    ],
    out_specs=pl.BlockSpec((BLK, HIDDEN), lambda b: (b, 0)),
    out_shape=jax.ShapeDtypeStruct((M_TC, HIDDEN), jnp.float32),
    compiler_params=pltpu.CompilerParams(
        dimension_semantics=("parallel",)),
)


def kernel(x, emb):
    pairs = [
        (emb[2 * p][:, None, :] + emb[2 * p + 1][None, :, :]).reshape(
            PAIRB, HIDDEN)
        for p in range(4)
    ]
    tbl = jnp.concatenate(pairs + [emb[8]], axis=0)
    x_sc = x[:M_SC]
    xc = x_sc.T.reshape(N_FEATS, NCHUNKS, C).transpose(1, 0, 2).reshape(-1)
    sc_out = _sc_call(xc, tbl)

    x_tc = x[M_SC:]
    xg = x_tc.reshape(NB_TC, BLK, N_FEATS).transpose(0, 2, 1).reshape(
        NB_TC * N_FEATS, 1, BLK)
    emb_flat = emb.reshape(N_FEATS * VOCAB, HIDDEN)
    tblp = jnp.pad(emb_flat.astype(jnp.bfloat16),
                   ((0, 1024 - N_FEATS * VOCAB), (0, 0)))
    tc_out = _tc_call(xg, tblp)
    return jnp.concatenate([sc_out, tc_out], axis=0)
